# baseline (speedup 1.0000x reference)
import functools

import jax
import jax.numpy as jnp
from jax import lax
from jax.experimental import pallas as pl
from jax.experimental.pallas import tpu as pltpu
from jax.experimental.pallas import tpu_sc as plsc

B, H, W, C = 4, 112, 112, 96
OUT_H, OUT_W = 2 * H, 2 * W
HW = H * W
OUT_HW = OUT_H * OUT_W
NC, NS, L = 2, 16, 16
NW = NC * NS
JOBS = B * C
JOBS_PER_W = JOBS // NW
VECS = HW // L
ZVECS = OUT_HW // L
_XK = jnp.int32(0x2AAAAAAA)


def _div96(m):
    t = m >> 5
    a = t >> 12
    b = t & 4095
    return a * 1365 + ((a + b) * 21846 >> 16)


_mesh = plsc.VectorSubcoreMesh(
    core_axis_name="c", subcore_axis_name="s", num_cores=NC, num_subcores=NS
)


SC_UNROLL = 8
Z_UNROLL = 16
CH = 2
CHE = HW // CH
CHV = CHE // L // SC_UNROLL
NCHUNKS = JOBS_PER_W * CH


@functools.partial(
    pl.kernel,
    out_type=jax.ShapeDtypeStruct((JOBS, OUT_HW), jnp.float32),
    mesh=_mesh,
    scratch_types=[
        pltpu.VMEM((2, CHE), jnp.int32),
        pltpu.VMEM((2, CHE), jnp.float32),
        pltpu.VMEM((OUT_HW,), jnp.float32),
        pltpu.VMEM((OUT_HW,), jnp.float32),
    ]
    + [pltpu.SemaphoreType.DMA] * 6,
    compiler_params=pltpu.CompilerParams(needs_layout_passes=False),
)
def _unpool_sc(mask_hbm, upd_hbm, out_hbm, mask_v, upd_v, acc0_v, acc1_v,
               sm0, sm1, su0, su1, so0, so1):
    wid = lax.axis_index("s") * NC + lax.axis_index("c")
    base = wid * JOBS_PER_W
    msems = (sm0, sm1)
    usems = (su0, su1)
    osems = (so0, so1)

    accs = (acc0_v, acc1_v)

    def zero_plane(a):
        ref = accs[a]

        def zero(i, c):
            b = i * (L * Z_UNROLL)
            for k in range(Z_UNROLL):
                ref[pl.ds(b + k * L, L)] = jnp.zeros((L,), jnp.float32)
            return c

        lax.fori_loop(0, ZVECS // Z_UNROLL, zero, 0)

    zero_plane(0)
    zero_plane(1)

    def fetch(g):
        t, c = g // CH, g % CH
        buf = g % 2
        off = c * CHE
        return (
            pltpu.async_copy(
                mask_hbm.at[base + t, pl.ds(off, CHE)], mask_v.at[buf], msems[buf]
            ),
            pltpu.async_copy(
                upd_hbm.at[base + t, pl.ds(off, CHE)], upd_v.at[buf], usems[buf]
            ),
        )

    pending = fetch(0)
    for t in range(JOBS_PER_W):
        a = t % 2
        o = 1 - a
        for c in range(CH):
            g = t * CH + c
            buf = g % 2
            for h in pending:
                h.wait()
            if g + 1 < NCHUNKS:
                pending = fetch(g + 1)

            def step(i, carry):
                b = i * (L * SC_UNROLL)
                for k in range(SC_UNROLL):
                    m = mask_v[buf, pl.ds(b + k * L, L)] ^ _XK
                    u = upd_v[buf, pl.ds(b + k * L, L)]
                    plsc.addupdate_scatter(accs[a], [_div96(m)], u)
                return carry

            lax.fori_loop(0, CHV, step, 0)

        if t >= 1:
            pltpu.make_async_copy(accs[o], out_hbm.at[base + t - 1], osems[o]).wait()
            if t + 1 < JOBS_PER_W:
                zero_plane(o)
        pltpu.async_copy(accs[a], out_hbm.at[base + t], osems[a])
    pltpu.make_async_copy(
        accs[(JOBS_PER_W - 1) % 2],
        out_hbm.at[base + JOBS_PER_W - 1],
        osems[(JOBS_PER_W - 1) % 2],
    ).wait()


@jax.jit
def kernel(updates, mask):
    mask_t = jnp.transpose(mask ^ _XK, (0, 3, 1, 2)).reshape(JOBS, HW)
    upd_t = jnp.transpose(updates + jnp.float32(0.0), (0, 3, 1, 2)).reshape(JOBS, HW)
    out_t = _unpool_sc(mask_t, upd_t)
    return jnp.transpose(out_t.reshape(B, C, OUT_H, OUT_W), (0, 2, 3, 1)) + jnp.float32(0.0)


# speedup vs baseline: 1.0452x; 1.0452x over previous
import functools

import jax
import jax.numpy as jnp
from jax import lax
from jax.experimental import pallas as pl
from jax.experimental.pallas import tpu as pltpu
from jax.experimental.pallas import tpu_sc as plsc

B, H, W, C = 4, 112, 112, 96
OUT_H, OUT_W = 2 * H, 2 * W
HW = H * W
OUT_HW = OUT_H * OUT_W
NC, NS, L = 2, 16, 16
NW = NC * NS
JOBS = B * C
JOBS_PER_W = JOBS // NW
VECS = HW // L
ZVECS = OUT_HW // L


def _div96(m):
    t = m >> 5
    a = t >> 12
    b = t & 4095
    return a * 1365 + ((a + b) * 21846 >> 16)


_mesh = plsc.VectorSubcoreMesh(
    core_axis_name="c", subcore_axis_name="s", num_cores=NC, num_subcores=NS
)


SC_UNROLL = 8
Z_UNROLL = 16
CH = 2
CHE = HW // CH
CHV = CHE // L // SC_UNROLL
NCHUNKS = JOBS_PER_W * CH


@functools.partial(
    pl.kernel,
    out_type=jax.ShapeDtypeStruct((JOBS, OUT_HW), jnp.float32),
    mesh=_mesh,
    scratch_types=[
        pltpu.VMEM((2, CHE), jnp.int32),
        pltpu.VMEM((2, CHE), jnp.float32),
        pltpu.VMEM((OUT_HW,), jnp.float32),
        pltpu.VMEM((OUT_HW,), jnp.float32),
    ]
    + [pltpu.SemaphoreType.DMA] * 6,
    compiler_params=pltpu.CompilerParams(needs_layout_passes=False),
)
def _unpool_sc(mask_hbm, upd_hbm, out_hbm, mask_v, upd_v, acc0_v, acc1_v,
               sm0, sm1, su0, su1, so0, so1):
    wid = lax.axis_index("s") * NC + lax.axis_index("c")
    base = wid * JOBS_PER_W
    msems = (sm0, sm1)
    usems = (su0, su1)
    osems = (so0, so1)

    accs = (acc0_v, acc1_v)

    def zero_plane(a):
        ref = accs[a]

        def zero(i, c):
            b = i * (L * Z_UNROLL)
            for k in range(Z_UNROLL):
                ref[pl.ds(b + k * L, L)] = jnp.zeros((L,), jnp.float32)
            return c

        lax.fori_loop(0, ZVECS // Z_UNROLL, zero, 0)

    zero_plane(0)
    zero_plane(1)

    def fetch(g):
        t, c = g // CH, g % CH
        buf = g % 2
        off = c * CHE
        return (
            pltpu.async_copy(
                mask_hbm.at[base + t, pl.ds(off, CHE)], mask_v.at[buf], msems[buf]
            ),
            pltpu.async_copy(
                upd_hbm.at[base + t, pl.ds(off, CHE)], upd_v.at[buf], usems[buf]
            ),
        )

    pending = fetch(0)
    for t in range(JOBS_PER_W):
        a = t % 2
        o = 1 - a
        for c in range(CH):
            g = t * CH + c
            buf = g % 2
            for h in pending:
                h.wait()
            if g + 1 < NCHUNKS:
                pending = fetch(g + 1)

            def step(i, carry):
                b = i * (L * SC_UNROLL)
                for k in range(SC_UNROLL):
                    m = mask_v[buf, pl.ds(b + k * L, L)]
                    u = upd_v[buf, pl.ds(b + k * L, L)]
                    plsc.addupdate_scatter(accs[a], [_div96(m)], u)
                return carry

            lax.fori_loop(0, CHV, step, 0)

        if t >= 1:
            pltpu.make_async_copy(accs[o], out_hbm.at[base + t - 1], osems[o]).wait()
            if t + 1 < JOBS_PER_W:
                zero_plane(o)
        pltpu.async_copy(accs[a], out_hbm.at[base + t], osems[a])
    pltpu.make_async_copy(
        accs[(JOBS_PER_W - 1) % 2],
        out_hbm.at[base + JOBS_PER_W - 1],
        osems[(JOBS_PER_W - 1) % 2],
    ).wait()


@jax.jit
def kernel(updates, mask):
    mask_t = jnp.transpose(mask.reshape(B, HW, C), (0, 2, 1)).reshape(JOBS, HW)
    upd_t = jnp.transpose(updates.reshape(B, HW, C), (0, 2, 1)).reshape(JOBS, HW)
    out_t = _unpool_sc(mask_t, upd_t)
    return jnp.transpose(out_t.reshape(B, C, OUT_HW), (0, 2, 1)).reshape(
        B, OUT_H, OUT_W, C
    )


# speedup vs baseline: 1.0462x; 1.0010x over previous
import functools

import jax
import jax.numpy as jnp
from jax import lax
from jax.experimental import pallas as pl
from jax.experimental import layout as jex_layout
from jax.experimental.pallas import tpu as pltpu
from jax.experimental.pallas import tpu_sc as plsc

B, H, W, C = 4, 112, 112, 96
OUT_H, OUT_W = 2 * H, 2 * W
HW = H * W
OUT_HW = OUT_H * OUT_W
NC, NS, L = 2, 16, 16
NW = NC * NS
JOBS = B * C
JOBS_PER_W = JOBS // NW
VECS = HW // L
ZVECS = OUT_HW // L


def _div96(m):
    t = m >> 5
    a = t >> 12
    b = t & 4095
    return a * 1365 + ((a + b) * 21846 >> 16)


_mesh = plsc.VectorSubcoreMesh(
    core_axis_name="c", subcore_axis_name="s", num_cores=NC, num_subcores=NS
)


SC_UNROLL = 8
Z_UNROLL = 16
CH = 2
CHE = HW // CH
CHV = CHE // L // SC_UNROLL
NCHUNKS = JOBS_PER_W * CH


@functools.partial(
    pl.kernel,
    out_type=jax.ShapeDtypeStruct((JOBS, OUT_HW), jnp.float32),
    mesh=_mesh,
    scratch_types=[
        pltpu.VMEM((2, CHE), jnp.int32),
        pltpu.VMEM((2, CHE), jnp.float32),
        pltpu.VMEM((OUT_HW,), jnp.float32),
        pltpu.VMEM((OUT_HW,), jnp.float32),
    ]
    + [pltpu.SemaphoreType.DMA] * 6,
    compiler_params=pltpu.CompilerParams(needs_layout_passes=False),
)
def _unpool_sc(mask_hbm, upd_hbm, out_hbm, mask_v, upd_v, acc0_v, acc1_v,
               sm0, sm1, su0, su1, so0, so1):
    wid = lax.axis_index("s") * NC + lax.axis_index("c")
    base = wid * JOBS_PER_W
    msems = (sm0, sm1)
    usems = (su0, su1)
    osems = (so0, so1)

    accs = (acc0_v, acc1_v)

    def zero_plane(a):
        ref = accs[a]

        def zero(i, c):
            b = i * (L * Z_UNROLL)
            for k in range(Z_UNROLL):
                ref[pl.ds(b + k * L, L)] = jnp.zeros((L,), jnp.float32)
            return c

        lax.fori_loop(0, ZVECS // Z_UNROLL, zero, 0)

    zero_plane(0)
    zero_plane(1)

    def fetch(g):
        t, c = g // CH, g % CH
        buf = g % 2
        off = c * CHE
        return (
            pltpu.async_copy(
                mask_hbm.at[base + t, pl.ds(off, CHE)], mask_v.at[buf], msems[buf]
            ),
            pltpu.async_copy(
                upd_hbm.at[base + t, pl.ds(off, CHE)], upd_v.at[buf], usems[buf]
            ),
        )

    pending = fetch(0)
    for t in range(JOBS_PER_W):
        a = t % 2
        o = 1 - a
        for c in range(CH):
            g = t * CH + c
            buf = g % 2
            for h in pending:
                h.wait()
            if g + 1 < NCHUNKS:
                pending = fetch(g + 1)

            def step(i, carry):
                b = i * (L * SC_UNROLL)
                for k in range(SC_UNROLL):
                    m = mask_v[buf, pl.ds(b + k * L, L)]
                    u = upd_v[buf, pl.ds(b + k * L, L)]
                    plsc.addupdate_scatter(accs[a], [_div96(m)], u)
                return carry

            lax.fori_loop(0, CHV, step, 0)

        if t >= 1:
            pltpu.make_async_copy(accs[o], out_hbm.at[base + t - 1], osems[o]).wait()
            if t + 1 < JOBS_PER_W:
                zero_plane(o)
        pltpu.async_copy(accs[a], out_hbm.at[base + t], osems[a])
    pltpu.make_async_copy(
        accs[(JOBS_PER_W - 1) % 2],
        out_hbm.at[base + JOBS_PER_W - 1],
        osems[(JOBS_PER_W - 1) % 2],
    ).wait()


def _kernel_impl(updates, mask):
    mask_t = jnp.transpose(mask.reshape(B, HW, C), (0, 2, 1)).reshape(JOBS, HW)
    upd_t = jnp.transpose(updates.reshape(B, HW, C), (0, 2, 1)).reshape(JOBS, HW)
    out_t = _unpool_sc(mask_t, upd_t)
    return jnp.transpose(out_t.reshape(B, C, OUT_HW), (0, 2, 1)).reshape(
        B, OUT_H, OUT_W, C
    )


_jitted = None


def kernel(updates, mask):
    global _jitted
    if _jitted is None:
        try:
            dev = next(iter(updates.devices()))
        except Exception:
            dev = jax.devices()[0]
        fmt = jex_layout.Format(
            jex_layout.Layout(major_to_minor=(0, 1, 2, 3)),
            jax.sharding.SingleDeviceSharding(dev),
        )
        _jitted = jax.jit(_kernel_impl, out_shardings=fmt)
    return _jitted(updates, mask)


# speedup vs baseline: 1.2742x; 1.2179x over previous
import functools

import jax
import jax.numpy as jnp
from jax import lax
from jax.experimental import pallas as pl
from jax.experimental.pallas import tpu as pltpu
from jax.experimental.pallas import tpu_sc as plsc

B, H, W, C = 4, 112, 112, 96
OUT_H, OUT_W = 2 * H, 2 * W
HW = H * W
OUT_HW = OUT_H * OUT_W
NC, NS, L = 2, 16, 16
NW = NC * NS
JOBS = B * C
JOBS_PER_W = JOBS // NW
VECS = HW // L
ZVECS = OUT_HW // L


def _div96(m):
    t = m >> 5
    a = t >> 12
    b = t & 4095
    return a * 1365 + ((a + b) * 21846 >> 16)


_mesh = plsc.VectorSubcoreMesh(
    core_axis_name="c", subcore_axis_name="s", num_cores=NC, num_subcores=NS
)


SC_UNROLL = 8
Z_UNROLL = 16
CH = 2
CHE = HW // CH
CHV = CHE // L // SC_UNROLL
NCHUNKS = JOBS_PER_W * CH


@functools.partial(
    pl.kernel,
    out_type=jax.ShapeDtypeStruct((JOBS, OUT_HW), jnp.float32),
    mesh=_mesh,
    scratch_types=[
        pltpu.VMEM((2, CHE), jnp.int32),
        pltpu.VMEM((2, CHE), jnp.float32),
        pltpu.VMEM((OUT_HW,), jnp.float32),
        pltpu.VMEM((OUT_HW,), jnp.float32),
    ]
    + [pltpu.SemaphoreType.DMA] * 6,
    compiler_params=pltpu.CompilerParams(needs_layout_passes=False),
)
def _unpool_sc(mask_hbm, upd_hbm, out_hbm, mask_v, upd_v, acc0_v, acc1_v,
               sm0, sm1, su0, su1, so0, so1):
    wid = lax.axis_index("s") * NC + lax.axis_index("c")
    base = wid * JOBS_PER_W
    msems = (sm0, sm1)
    usems = (su0, su1)
    osems = (so0, so1)

    accs = (acc0_v, acc1_v)

    def zero_plane(a):
        ref = accs[a]

        @plsc.parallel_loop(0, ZVECS // Z_UNROLL)
        def zero(i):
            b = i * (L * Z_UNROLL)
            for k in range(Z_UNROLL):
                ref[pl.ds(b + k * L, L)] = jnp.zeros((L,), jnp.float32)

    zero_plane(0)
    zero_plane(1)

    def fetch(g):
        t, c = g // CH, g % CH
        buf = g % 2
        off = c * CHE
        return (
            pltpu.async_copy(
                mask_hbm.at[base + t, pl.ds(off, CHE)], mask_v.at[buf], msems[buf]
            ),
            pltpu.async_copy(
                upd_hbm.at[base + t, pl.ds(off, CHE)], upd_v.at[buf], usems[buf]
            ),
        )

    pending = fetch(0)
    for t in range(JOBS_PER_W):
        a = t % 2
        o = 1 - a
        for c in range(CH):
            g = t * CH + c
            buf = g % 2
            for h in pending:
                h.wait()
            if g + 1 < NCHUNKS:
                pending = fetch(g + 1)

            @plsc.parallel_loop(0, CHV)
            def step(i):
                b = i * (L * SC_UNROLL)
                for k in range(SC_UNROLL):
                    m = mask_v[buf, pl.ds(b + k * L, L)]
                    u = upd_v[buf, pl.ds(b + k * L, L)]
                    plsc.addupdate_scatter(accs[a], [_div96(m)], u)

        if t >= 1:
            pltpu.make_async_copy(accs[o], out_hbm.at[base + t - 1], osems[o]).wait()
            if t + 1 < JOBS_PER_W:
                zero_plane(o)
        pltpu.async_copy(accs[a], out_hbm.at[base + t], osems[a])
    pltpu.make_async_copy(
        accs[(JOBS_PER_W - 1) % 2],
        out_hbm.at[base + JOBS_PER_W - 1],
        osems[(JOBS_PER_W - 1) % 2],
    ).wait()


@jax.jit
def kernel(updates, mask):
    mask_t = jnp.transpose(mask.reshape(B, HW, C), (0, 2, 1)).reshape(JOBS, HW)
    upd_t = jnp.transpose(updates.reshape(B, HW, C), (0, 2, 1)).reshape(JOBS, HW)
    out_t = _unpool_sc(mask_t, upd_t)
    return jnp.transpose(out_t.reshape(B, C, OUT_HW), (0, 2, 1)).reshape(
        B, OUT_H, OUT_W, C
    )


# speedup vs baseline: 1.3058x; 1.0248x over previous
import functools

import jax
import jax.numpy as jnp
from jax import lax
from jax.experimental import pallas as pl
from jax.experimental.pallas import tpu as pltpu
from jax.experimental.pallas import tpu_sc as plsc

B, H, W, C = 4, 112, 112, 96
OUT_H, OUT_W = 2 * H, 2 * W
HW = H * W
OUT_HW = OUT_H * OUT_W
NC, NS, L = 2, 16, 16
NW = NC * NS
JOBS = B * C
JOBS_PER_W = JOBS // NW
VECS = HW // L
ZVECS = OUT_HW // L


def _div96(m):
    t = m >> 5
    a = t >> 12
    b = t & 4095
    return a * 1365 + ((a + b) * 21846 >> 16)


_mesh = plsc.VectorSubcoreMesh(
    core_axis_name="c", subcore_axis_name="s", num_cores=NC, num_subcores=NS
)


SC_UNROLL = 8
Z_UNROLL = 16
CH = 2
CHE = HW // CH
CHV = CHE // L // SC_UNROLL
NCHUNKS = JOBS_PER_W * CH


@functools.partial(
    pl.kernel,
    out_type=jax.ShapeDtypeStruct((JOBS, OUT_HW), jnp.float32),
    mesh=_mesh,
    scratch_types=[
        pltpu.VMEM((2, CHE), jnp.int32),
        pltpu.VMEM((2, CHE), jnp.float32),
        pltpu.VMEM((OUT_HW,), jnp.float32),
        pltpu.VMEM((OUT_HW,), jnp.float32),
    ]
    + [pltpu.SemaphoreType.DMA] * 6,
    compiler_params=pltpu.CompilerParams(needs_layout_passes=False),
)
def _unpool_sc(mask_hbm, upd_hbm, out_hbm, mask_v, upd_v, acc0_v, acc1_v,
               sm0, sm1, su0, su1, so0, so1):
    wid = lax.axis_index("s") * NC + lax.axis_index("c")
    base = wid * JOBS_PER_W
    msems = (sm0, sm1)
    usems = (su0, su1)
    osems = (so0, so1)

    accs = (acc0_v, acc1_v)

    def zero_plane(a):
        ref = accs[a]

        @plsc.parallel_loop(0, ZVECS, unroll=Z_UNROLL)
        def zero(i):
            ref[pl.ds(i * L, L)] = jnp.zeros((L,), jnp.float32)

    zero_plane(0)
    zero_plane(1)

    def fetch(g):
        t, c = g // CH, g % CH
        buf = g % 2
        off = c * CHE
        return (
            pltpu.async_copy(
                mask_hbm.at[base + t, pl.ds(off, CHE)], mask_v.at[buf], msems[buf]
            ),
            pltpu.async_copy(
                upd_hbm.at[base + t, pl.ds(off, CHE)], upd_v.at[buf], usems[buf]
            ),
        )

    pending = fetch(0)
    for t in range(JOBS_PER_W):
        a = t % 2
        o = 1 - a
        for c in range(CH):
            g = t * CH + c
            buf = g % 2
            for h in pending:
                h.wait()
            if g + 1 < NCHUNKS:
                pending = fetch(g + 1)

            @plsc.parallel_loop(0, CHE // L, unroll=SC_UNROLL)
            def step(i):
                m = mask_v[buf, pl.ds(i * L, L)]
                u = upd_v[buf, pl.ds(i * L, L)]
                plsc.addupdate_scatter(accs[a], [_div96(m)], u)

        if t >= 1:
            pltpu.make_async_copy(accs[o], out_hbm.at[base + t - 1], osems[o]).wait()
            if t + 1 < JOBS_PER_W:
                zero_plane(o)
        pltpu.async_copy(accs[a], out_hbm.at[base + t], osems[a])
    pltpu.make_async_copy(
        accs[(JOBS_PER_W - 1) % 2],
        out_hbm.at[base + JOBS_PER_W - 1],
        osems[(JOBS_PER_W - 1) % 2],
    ).wait()


@jax.jit
def kernel(updates, mask):
    mask_t = jnp.transpose(mask.reshape(B, HW, C), (0, 2, 1)).reshape(JOBS, HW)
    upd_t = jnp.transpose(updates.reshape(B, HW, C), (0, 2, 1)).reshape(JOBS, HW)
    out_t = _unpool_sc(mask_t, upd_t)
    return jnp.transpose(out_t.reshape(B, C, OUT_HW), (0, 2, 1)).reshape(
        B, OUT_H, OUT_W, C
    )
